# Initial kernel scaffold; baseline (speedup 1.0000x reference)
import jax
import jax.numpy as jnp
from jax.experimental import pallas as pl


def kernel(x, edge_index, W_self, b_self, W_s2d, b_s2d, W_d2s, b_d2s):
    raise NotImplementedError("write your pallas kernel here")


# speedup vs baseline: 5.4705x; 5.4705x over previous
import functools

import jax
import jax.numpy as jnp
from jax import lax
from jax.experimental import pallas as pl
from jax.experimental.pallas import tpu as pltpu
from jax.experimental.pallas import tpu_sc as plsc

_N = 10000
_E = 320000
_D = 128
_H = _D // 2
_ALPHA = 0.5

_NC = 2
_NS = 16
_C = 80
_EPT = _E // _NS
_CHUNKS = _EPT // _C
_RPT = 624
_RTAIL = _N - _NS * _RPT


def _sc_aggregate(x0, x1, idx, zer_nh, zer_n):
    mesh = plsc.VectorSubcoreMesh(core_axis_name="c", subcore_axis_name="s")

    @functools.partial(
        pl.kernel,
        out_type=(
            jax.ShapeDtypeStruct((_NC, 2, _N, _H), jnp.float32),
            jax.ShapeDtypeStruct((_NC, _N), jnp.float32),
        ),
        mesh=mesh,
        compiler_params=pltpu.CompilerParams(use_tc_tiling_on_sc=False),
        scratch_types=[
            pltpu.VMEM((_CHUNKS, _C), jnp.int32),
            pltpu.VMEM((_CHUNKS, _C), jnp.int32),
            pltpu.VMEM((_C, _H), jnp.float32),
            pltpu.VMEM((_C,), jnp.float32),
            pltpu.VMEM_SHARED((_N, _H), jnp.float32),
            pltpu.VMEM_SHARED((_N,), jnp.float32),
            pltpu.SemaphoreType.DMA,
        ],
    )
    def k(x0_hbm, x1_hbm, idx_hbm, zer_nh_hbm, zer_n_hbm,
          acc_out, cnt_out, gslab, sslab, rows_v, ones_v, acc_sh, cnt_sh, sem):
        cid = lax.axis_index("c")
        sid = lax.axis_index("s")

        pltpu.sync_copy(idx_hbm.at[cid, sid], gslab)
        pltpu.sync_copy(idx_hbm.at[1 - cid, sid], sslab)

        for j in range(_C // 16):
            ones_v[pl.ds(16 * j, 16)] = jnp.ones((16,), jnp.float32)

        r0 = sid * _RPT

        for p, xp_hbm in enumerate((x0_hbm, x1_hbm)):
            pltpu.sync_copy(zer_nh_hbm.at[pl.ds(r0, _RPT)],
                            acc_sh.at[pl.ds(r0, _RPT)])

            @pl.when(sid == 0)
            def _():
                pltpu.sync_copy(zer_nh_hbm.at[pl.ds(_NS * _RPT, _RTAIL)],
                                acc_sh.at[pl.ds(_NS * _RPT, _RTAIL)])
                if p == 0:
                    pltpu.sync_copy(zer_n_hbm, cnt_sh)

            plsc.subcore_barrier()

            def body(kk, _):
                gidx = gslab.at[kk]
                sidx = sslab.at[kk]
                pltpu.async_copy(xp_hbm.at[gidx], rows_v, sem).wait()
                pltpu.sync_copy(rows_v, acc_sh.at[sidx], add=True)
                if p == 0:
                    pltpu.sync_copy(ones_v, cnt_sh.at[sidx], add=True)
                return 0

            lax.fori_loop(0, _CHUNKS, body, 0)

            plsc.subcore_barrier()

            pltpu.sync_copy(acc_sh.at[pl.ds(r0, _RPT)],
                            acc_out.at[cid, p, pl.ds(r0, _RPT)])

            @pl.when(sid == 0)
            def _():
                pltpu.sync_copy(acc_sh.at[pl.ds(_NS * _RPT, _RTAIL)],
                                acc_out.at[cid, p, pl.ds(_NS * _RPT, _RTAIL)])
                if p == 0:
                    pltpu.sync_copy(cnt_sh, cnt_out.at[cid])

            plsc.subcore_barrier()

    return k(x0, x1, idx, zer_nh, zer_n)


_BLK = 1000


def _tc_body(x_ref, a00_ref, a01_ref, a10_ref, a11_ref, c0_ref, c1_ref,
             wm_ref, w0a_ref, w0b_ref, w1a_ref, w1b_ref, b_ref, o_ref):
    f32 = jnp.float32
    r0 = 1.0 / jnp.maximum(c0_ref[...], 1.0)
    r1 = 1.0 / jnp.maximum(c1_ref[...], 1.0)
    o = jnp.dot(x_ref[...], wm_ref[...], preferred_element_type=f32)
    o += jnp.dot(a00_ref[...] * r0, w0a_ref[...], preferred_element_type=f32)
    o += jnp.dot(a01_ref[...] * r0, w0b_ref[...], preferred_element_type=f32)
    o += jnp.dot(a10_ref[...] * r1, w1a_ref[...], preferred_element_type=f32)
    o += jnp.dot(a11_ref[...] * r1, w1b_ref[...], preferred_element_type=f32)
    o_ref[...] = o + b_ref[...]


def _tc_combine(x, a00, a01, a10, a11, c0, c1, wm, w0a, w0b, w1a, w1b, b):
    grid = (_N // _BLK,)
    half = pl.BlockSpec((_BLK, _H), lambda i: (i, 0))
    full = pl.BlockSpec((_BLK, _D), lambda i: (i, 0))
    col = pl.BlockSpec((_BLK, 1), lambda i: (i, 0))
    wfull = pl.BlockSpec((_D, _D), lambda i: (0, 0))
    whalf = pl.BlockSpec((_H, _D), lambda i: (0, 0))
    brow = pl.BlockSpec((1, _D), lambda i: (0, 0))
    return pl.pallas_call(
        _tc_body,
        grid=grid,
        in_specs=[full, half, half, half, half, col, col,
                  wfull, whalf, whalf, whalf, whalf, brow],
        out_specs=full,
        out_shape=jax.ShapeDtypeStruct((_N, _D), jnp.float32),
    )(x, a00, a01, a10, a11, c0, c1, wm, w0a, w0b, w1a, w1b, b)


def kernel(x, edge_index, W_self, b_self, W_s2d, b_s2d, W_d2s, b_d2s):
    idx = edge_index.reshape(_NC, _NS, _CHUNKS, _C)
    x0 = x[:, :_H]
    x1 = x[:, _H:]
    zer_nh = jnp.zeros((_N, _H), jnp.float32)
    zer_n = jnp.zeros((_N,), jnp.float32)

    acc, cnt = _sc_aggregate(x0, x1, idx, zer_nh, zer_n)

    wm = W_self.T
    w0 = (1.0 - _ALPHA) * W_s2d.T
    w1 = _ALPHA * W_d2s.T
    b = (b_self + (1.0 - _ALPHA) * b_s2d + _ALPHA * b_d2s).reshape(1, _D)
    return _tc_combine(x, acc[0, 0], acc[0, 1], acc[1, 0], acc[1, 1],
                       cnt[0].reshape(_N, 1), cnt[1].reshape(_N, 1),
                       wm, w0[:_H], w0[_H:], w1[:_H], w1[_H:], b)


# speedup vs baseline: 8.3102x; 1.5191x over previous
import functools

import jax
import jax.numpy as jnp
from jax import lax
from jax.experimental import pallas as pl
from jax.experimental.pallas import tpu as pltpu
from jax.experimental.pallas import tpu_sc as plsc

_N = 10000
_E = 320000
_D = 128
_H = _D // 2
_ALPHA = 0.5

_NC = 2
_NS = 16
_C = 80
_EPT = _E // _NS
_CHUNKS = _EPT // _C
_RPT = 624
_RTAIL = _N - _NS * _RPT


def _sc_aggregate(x0, x1, idx, zer_nh, zer_n):
    mesh = plsc.VectorSubcoreMesh(core_axis_name="c", subcore_axis_name="s")

    @functools.partial(
        pl.kernel,
        out_type=(
            jax.ShapeDtypeStruct((_NC, 2, _N, _H), jnp.float32),
            jax.ShapeDtypeStruct((_NC, _N), jnp.float32),
        ),
        mesh=mesh,
        compiler_params=pltpu.CompilerParams(use_tc_tiling_on_sc=False),
        scratch_types=[
            pltpu.VMEM((_CHUNKS, _C), jnp.int32),
            pltpu.VMEM((_CHUNKS, _C), jnp.int32),
            pltpu.VMEM((_C, _H), jnp.float32),
            pltpu.VMEM((_C, _H), jnp.float32),
            pltpu.VMEM((_C,), jnp.float32),
            pltpu.VMEM_SHARED((_N, _H), jnp.float32),
            pltpu.VMEM_SHARED((_N,), jnp.float32),
            pltpu.SemaphoreType.DMA,
            pltpu.SemaphoreType.DMA,
            pltpu.SemaphoreType.DMA,
            pltpu.SemaphoreType.DMA,
            pltpu.SemaphoreType.DMA,
        ],
    )
    def k(x0_hbm, x1_hbm, idx_hbm, zer_nh_hbm, zer_n_hbm,
          acc_out, cnt_out, gslab, sslab, rows0, rows1, ones_v, acc_sh, cnt_sh,
          sg0, sg1, ss0, ss1, scn):
        cid = lax.axis_index("c")
        sid = lax.axis_index("s")

        pltpu.sync_copy(idx_hbm.at[cid, sid], gslab)
        pltpu.sync_copy(idx_hbm.at[1 - cid, sid], sslab)

        for j in range(_C // 16):
            ones_v[pl.ds(16 * j, 16)] = jnp.ones((16,), jnp.float32)

        r0 = sid * _RPT

        for p, xp_hbm in enumerate((x0_hbm, x1_hbm)):
            pltpu.sync_copy(zer_nh_hbm.at[pl.ds(r0, _RPT)],
                            acc_sh.at[pl.ds(r0, _RPT)])

            @pl.when(sid == 0)
            def _():
                pltpu.sync_copy(zer_nh_hbm.at[pl.ds(_NS * _RPT, _RTAIL)],
                                acc_sh.at[pl.ds(_NS * _RPT, _RTAIL)])
                if p == 0:
                    pltpu.sync_copy(zer_n_hbm, cnt_sh)

            plsc.subcore_barrier()

            def g_start(kk, rows, sg):
                pltpu.async_copy(xp_hbm.at[gslab.at[kk]], rows, sg)

            def g_wait(kk, rows, sg):
                pltpu.make_async_copy(xp_hbm.at[gslab.at[kk]], rows, sg).wait()

            def s_start(kk, rows, ss):
                pltpu.async_copy(rows, acc_sh.at[sslab.at[kk]], ss, add=True)

            def s_wait(kk, rows, ss):
                pltpu.make_async_copy(rows, acc_sh.at[sslab.at[kk]], ss).wait()

            def c_start(kk):
                pltpu.async_copy(ones_v, cnt_sh.at[sslab.at[kk]], scn, add=True)

            def c_wait(kk):
                pltpu.make_async_copy(ones_v, cnt_sh.at[sslab.at[kk]], scn).wait()

            g_start(0, rows0, sg0)
            g_start(1, rows1, sg1)

            def body(i, _):
                k0 = 2 * i
                k1 = k0 + 1
                g_wait(k0, rows0, sg0)
                s_start(k0, rows0, ss0)
                g_wait(k1, rows1, sg1)
                s_start(k1, rows1, ss1)
                if p == 0:
                    @pl.when(i > 0)
                    def _():
                        c_wait(k0)
                        c_wait(k1)
                    c_start(k0)
                    c_start(k1)
                s_wait(k0, rows0, ss0)
                g_start(k0 + 2, rows0, sg0)
                s_wait(k1, rows1, ss1)
                g_start(k1 + 2, rows1, sg1)
                return 0

            lax.fori_loop(0, _CHUNKS // 2 - 1, body, 0)

            kl0, kl1 = _CHUNKS - 2, _CHUNKS - 1
            g_wait(kl0, rows0, sg0)
            s_start(kl0, rows0, ss0)
            g_wait(kl1, rows1, sg1)
            s_start(kl1, rows1, ss1)
            if p == 0:
                c_start(kl0)
                c_start(kl1)
                for _i in range(4):
                    c_wait(kl0)
            s_wait(kl0, rows0, ss0)
            s_wait(kl1, rows1, ss1)

            plsc.subcore_barrier()

            pltpu.sync_copy(acc_sh.at[pl.ds(r0, _RPT)],
                            acc_out.at[cid, p, pl.ds(r0, _RPT)])

            @pl.when(sid == 0)
            def _():
                pltpu.sync_copy(acc_sh.at[pl.ds(_NS * _RPT, _RTAIL)],
                                acc_out.at[cid, p, pl.ds(_NS * _RPT, _RTAIL)])
                if p == 0:
                    pltpu.sync_copy(cnt_sh, cnt_out.at[cid])

            plsc.subcore_barrier()

    return k(x0, x1, idx, zer_nh, zer_n)


_BLK = 1000


def _tc_body(x_ref, a00_ref, a01_ref, a10_ref, a11_ref, c0_ref, c1_ref,
             wm_ref, w0a_ref, w0b_ref, w1a_ref, w1b_ref, b_ref, o_ref):
    f32 = jnp.float32
    r0 = 1.0 / jnp.maximum(c0_ref[...], 1.0)
    r1 = 1.0 / jnp.maximum(c1_ref[...], 1.0)
    o = jnp.dot(x_ref[...], wm_ref[...], preferred_element_type=f32)
    o += jnp.dot(a00_ref[...] * r0, w0a_ref[...], preferred_element_type=f32)
    o += jnp.dot(a01_ref[...] * r0, w0b_ref[...], preferred_element_type=f32)
    o += jnp.dot(a10_ref[...] * r1, w1a_ref[...], preferred_element_type=f32)
    o += jnp.dot(a11_ref[...] * r1, w1b_ref[...], preferred_element_type=f32)
    o_ref[...] = o + b_ref[...]


def _tc_combine(x, a00, a01, a10, a11, c0, c1, wm, w0a, w0b, w1a, w1b, b):
    grid = (_N // _BLK,)
    half = pl.BlockSpec((_BLK, _H), lambda i: (i, 0))
    full = pl.BlockSpec((_BLK, _D), lambda i: (i, 0))
    col = pl.BlockSpec((_BLK, 1), lambda i: (i, 0))
    wfull = pl.BlockSpec((_D, _D), lambda i: (0, 0))
    whalf = pl.BlockSpec((_H, _D), lambda i: (0, 0))
    brow = pl.BlockSpec((1, _D), lambda i: (0, 0))
    return pl.pallas_call(
        _tc_body,
        grid=grid,
        in_specs=[full, half, half, half, half, col, col,
                  wfull, whalf, whalf, whalf, whalf, brow],
        out_specs=full,
        out_shape=jax.ShapeDtypeStruct((_N, _D), jnp.float32),
    )(x, a00, a01, a10, a11, c0, c1, wm, w0a, w0b, w1a, w1b, b)


def kernel(x, edge_index, W_self, b_self, W_s2d, b_s2d, W_d2s, b_d2s):
    idx = edge_index.reshape(_NC, _NS, _CHUNKS, _C)
    x0 = x[:, :_H]
    x1 = x[:, _H:]
    zer_nh = jnp.zeros((_N, _H), jnp.float32)
    zer_n = jnp.zeros((_N,), jnp.float32)

    acc, cnt = _sc_aggregate(x0, x1, idx, zer_nh, zer_n)

    wm = W_self.T
    w0 = (1.0 - _ALPHA) * W_s2d.T
    w1 = _ALPHA * W_d2s.T
    b = (b_self + (1.0 - _ALPHA) * b_s2d + _ALPHA * b_d2s).reshape(1, _D)
    return _tc_combine(x, acc[0, 0], acc[0, 1], acc[1, 0], acc[1, 1],
                       cnt[0].reshape(_N, 1), cnt[1].reshape(_N, 1),
                       wm, w0[:_H], w0[_H:], w1[:_H], w1[_H:], b)
